# baseline (speedup 1.0000x reference)
import functools

import jax
import jax.numpy as jnp
from jax import lax
from jax.experimental import pallas as pl
from jax.experimental.pallas import tpu as pltpu
from jax.experimental.pallas import tpu_sc as plsc

HIDDEN = 768
EPS = 1e-12
B, S = 4, 2048

NSPLIT = 4
BG = B // NSPLIT
TG = BG * S

NW = 32
T_W = TG // NW
K = 64
N_CHUNK = T_W // K


def _sc_body(ids_hbm, word_hbm, out_hbm, ids_all, wrows, gsem, osem):
    wid = lax.axis_index("s") * 2 + lax.axis_index("c")
    base = wid * T_W

    pltpu.sync_copy(ids_hbm.at[pl.ds(base, T_W)], ids_all)

    def wslot(rb):
        return wrows.at[pl.ds(rb * K, K)]

    def issue_gather(j, rb):
        pltpu.async_copy(word_hbm.at[ids_all.at[pl.ds(j * K, K)]],
                         wslot(rb), gsem.at[rb])

    def wait_gather(j, rb):
        pltpu.make_async_copy(word_hbm.at[ids_all.at[pl.ds(j * K, K)]],
                              wslot(rb), gsem.at[rb]).wait()

    def issue_wb(j, rb):
        pltpu.async_copy(wslot(rb), out_hbm.at[pl.ds(base + j * K, K)],
                         osem.at[rb])

    def wait_wb(j, rb):
        pltpu.make_async_copy(wslot(rb), out_hbm.at[pl.ds(base + j * K, K)],
                              osem.at[rb]).wait()

    issue_gather(0, 0)

    def ring(j, _):
        rb = j & 1

        @pl.when(j >= 1)
        def _():
            wait_wb(j - 1, 1 - rb)

        @pl.when(j < N_CHUNK - 1)
        def _():
            issue_gather(j + 1, 1 - rb)

        wait_gather(j, rb)
        issue_wb(j, rb)
        return 0
    lax.fori_loop(0, N_CHUNK, ring, 0)

    wait_wb(N_CHUNK - 1, (N_CHUNK - 1) & 1)


def _sc_gather(ids_group, word_table):
    mesh = plsc.VectorSubcoreMesh(core_axis_name="c", subcore_axis_name="s")
    f = pl.kernel(
        _sc_body,
        out_type=jax.ShapeDtypeStruct((TG, HIDDEN), jnp.float32),
        mesh=mesh,
        compiler_params=pltpu.CompilerParams(needs_layout_passes=False),
        scratch_types=[
            pltpu.VMEM((T_W,), jnp.int32),
            pltpu.VMEM((2 * K, HIDDEN), jnp.float32),
            pltpu.SemaphoreType.DMA((2,)),
            pltpu.SemaphoreType.DMA((2,)),
        ],
    )
    return f(ids_group, word_table)


def _tc_body(xg_ref, pos_ref, type_ref, ttf_ref, g_ref, b_ref, *rest):
    out_ref = rest[-1]
    x = xg_ref[...]
    pos = pos_ref[...]
    t0 = type_ref[0:1, :]
    dt = type_ref[1:2, :] - t0
    ttf = ttf_ref[...]
    y = x + pos + t0 + ttf * dt
    mean = jnp.mean(y, axis=-1, keepdims=True)
    var = jnp.mean(y * y, axis=-1, keepdims=True) - mean * mean
    rstd = lax.rsqrt(var + EPS)
    out_ref[...] = (y - mean) * rstd * g_ref[...] + b_ref[...]


def _tc_ln(group, xg, ttf_group, pos_table, type_table, gamma, beta, outbuf):
    in_specs = [
        pl.BlockSpec((S, HIDDEN), lambda j: (j, 0)),
        pl.BlockSpec((S, HIDDEN), lambda j: (0, 0)),
        pl.BlockSpec((2, HIDDEN), lambda j: (0, 0)),
        pl.BlockSpec((S, 1), lambda j: (j, 0)),
        pl.BlockSpec((1, HIDDEN), lambda j: (0, 0)),
        pl.BlockSpec((1, HIDDEN), lambda j: (0, 0)),
    ]
    args = [xg, pos_table, type_table, ttf_group, gamma, beta]
    aliases = {}
    if outbuf is not None:
        in_specs.append(pl.BlockSpec((S, HIDDEN),
                                     lambda j, g=group: (g * BG + j, 0)))
        args.append(outbuf)
        aliases = {6: 0}
    f = pl.pallas_call(
        _tc_body,
        grid=(BG,),
        in_specs=in_specs,
        out_specs=pl.BlockSpec((S, HIDDEN),
                               lambda j, g=group: (g * BG + j, 0)),
        out_shape=jax.ShapeDtypeStruct((B * S, HIDDEN), jnp.float32),
        input_output_aliases=aliases,
        compiler_params=pltpu.CompilerParams(
            dimension_semantics=("arbitrary",)),
    )
    return f(*args)


@jax.jit
def _emb(ids, ttf, word_table, pos_table, type_table, gamma, beta):
    gamma = gamma.reshape(1, HIDDEN)
    beta = beta.reshape(1, HIDDEN)
    out = None
    for g in range(NSPLIT):
        xg = _sc_gather(lax.dynamic_slice_in_dim(ids, g * TG, TG),
                        word_table)
        out = _tc_ln(g, xg,
                     lax.dynamic_slice_in_dim(ttf, g * TG, TG),
                     pos_table, type_table, gamma, beta, out)
    return out


def kernel(input_ids, token_type_ids, word_table, pos_table, type_table,
           gamma, beta):
    ids = input_ids.reshape(-1).astype(jnp.int32)
    ttf = token_type_ids.reshape(-1, 1).astype(jnp.float32)
    out = _emb(ids, ttf, word_table, pos_table, type_table, gamma, beta)
    return out.reshape(input_ids.shape[0], input_ids.shape[1], HIDDEN)


# speedup vs baseline: 1.2630x; 1.2630x over previous
import functools

import jax
import jax.numpy as jnp
from jax import lax
from jax.experimental import pallas as pl
from jax.experimental.pallas import tpu as pltpu
from jax.experimental.pallas import tpu_sc as plsc

HIDDEN = 768
EPS = 1e-12
B, S = 4, 2048

NSPLIT = 2
BG = B // NSPLIT
TG = BG * S

NW = 32
T_W = TG // NW
K = 64
N_CHUNK = T_W // K


def _sc_body(ids_hbm, word_hbm, out_hbm, ids_all, wrows, gsem, osem):
    wid = lax.axis_index("s") * 2 + lax.axis_index("c")
    base = wid * T_W

    pltpu.sync_copy(ids_hbm.at[pl.ds(base, T_W)], ids_all)

    def wslot(rb):
        return wrows.at[pl.ds(rb * K, K)]

    def issue_gather(j, rb):
        pltpu.async_copy(word_hbm.at[ids_all.at[pl.ds(j * K, K)]],
                         wslot(rb), gsem.at[rb])

    def wait_gather(j, rb):
        pltpu.make_async_copy(word_hbm.at[ids_all.at[pl.ds(j * K, K)]],
                              wslot(rb), gsem.at[rb]).wait()

    def issue_wb(j, rb):
        pltpu.async_copy(wslot(rb), out_hbm.at[pl.ds(base + j * K, K)],
                         osem.at[rb])

    def wait_wb(j, rb):
        pltpu.make_async_copy(wslot(rb), out_hbm.at[pl.ds(base + j * K, K)],
                              osem.at[rb]).wait()

    issue_gather(0, 0)

    def ring(j, _):
        rb = j & 1

        @pl.when(j >= 1)
        def _():
            wait_wb(j - 1, 1 - rb)

        @pl.when(j < N_CHUNK - 1)
        def _():
            issue_gather(j + 1, 1 - rb)

        wait_gather(j, rb)
        issue_wb(j, rb)
        return 0
    lax.fori_loop(0, N_CHUNK, ring, 0)

    wait_wb(N_CHUNK - 1, (N_CHUNK - 1) & 1)


def _sc_gather(ids_group, word_table):
    mesh = plsc.VectorSubcoreMesh(core_axis_name="c", subcore_axis_name="s")
    f = pl.kernel(
        _sc_body,
        out_type=jax.ShapeDtypeStruct((TG, HIDDEN), jnp.float32),
        mesh=mesh,
        compiler_params=pltpu.CompilerParams(needs_layout_passes=False),
        scratch_types=[
            pltpu.VMEM((T_W,), jnp.int32),
            pltpu.VMEM((2 * K, HIDDEN), jnp.float32),
            pltpu.SemaphoreType.DMA((2,)),
            pltpu.SemaphoreType.DMA((2,)),
        ],
    )
    return f(ids_group, word_table)


def _tc_body(xg_ref, pos_ref, type_ref, ttf_ref, g_ref, b_ref, *rest):
    out_ref = rest[-1]
    x = xg_ref[...]
    pos = pos_ref[...]
    t0 = type_ref[0:1, :]
    dt = type_ref[1:2, :] - t0
    ttf = ttf_ref[...]
    y = x + pos + t0 + ttf * dt
    mean = jnp.mean(y, axis=-1, keepdims=True)
    var = jnp.mean(y * y, axis=-1, keepdims=True) - mean * mean
    rstd = lax.rsqrt(var + EPS)
    out_ref[...] = (y - mean) * rstd * g_ref[...] + b_ref[...]


def _tc_ln(group, xg, ttf_group, pos_table, type_table, gamma, beta, outbuf):
    in_specs = [
        pl.BlockSpec((S, HIDDEN), lambda j: (j, 0)),
        pl.BlockSpec((S, HIDDEN), lambda j: (0, 0)),
        pl.BlockSpec((2, HIDDEN), lambda j: (0, 0)),
        pl.BlockSpec((S, 1), lambda j: (j, 0)),
        pl.BlockSpec((1, HIDDEN), lambda j: (0, 0)),
        pl.BlockSpec((1, HIDDEN), lambda j: (0, 0)),
    ]
    args = [xg, pos_table, type_table, ttf_group, gamma, beta]
    aliases = {}
    if outbuf is not None:
        in_specs.append(pl.BlockSpec((8, 128), lambda j: (0, 0)))
        args.append(outbuf)
        aliases = {6: 0}
    f = pl.pallas_call(
        _tc_body,
        grid=(BG,),
        in_specs=in_specs,
        out_specs=pl.BlockSpec((S, HIDDEN),
                               lambda j, g=group: (g * BG + j, 0)),
        out_shape=jax.ShapeDtypeStruct((B * S, HIDDEN), jnp.float32),
        input_output_aliases=aliases,
        compiler_params=pltpu.CompilerParams(
            dimension_semantics=("arbitrary",)),
    )
    return f(*args)


@jax.jit
def _emb(ids, ttf, word_table, pos_table, type_table, gamma, beta):
    gamma = gamma.reshape(1, HIDDEN)
    beta = beta.reshape(1, HIDDEN)
    out = None
    for g in range(NSPLIT):
        xg = _sc_gather(lax.dynamic_slice_in_dim(ids, g * TG, TG),
                        word_table)
        out = _tc_ln(g, xg,
                     lax.dynamic_slice_in_dim(ttf, g * TG, TG),
                     pos_table, type_table, gamma, beta, out)
    return out


def kernel(input_ids, token_type_ids, word_table, pos_table, type_table,
           gamma, beta):
    ids = input_ids.reshape(-1).astype(jnp.int32)
    ttf = token_type_ids.reshape(-1, 1).astype(jnp.float32)
    out = _emb(ids, ttf, word_table, pos_table, type_table, gamma, beta)
    return out.reshape(input_ids.shape[0], input_ids.shape[1], HIDDEN)
